# baseline (speedup 1.0000x reference)
import functools

import jax
import jax.numpy as jnp
from jax import lax
from jax.experimental import pallas as pl
from jax.experimental.pallas import tpu as pltpu

NUM_SLOTS = 4096
HIDDEN = 4096
T_TOK = 8192
KS = 8
EMA_ALPHA = 0.1
WRITE_TOP_K = 2048

K1_BLK = 256
K1_GRID = T_TOK // K1_BLK


def _imp_kernel(hs_ref, attn_ref, w_ref, b_ref, out_ref):
    hs = hs_ref[...]
    ssq = jnp.sum(hs * hs, axis=-1)
    magnitude = jnp.sqrt(ssq)
    a = attn_ref[0]
    entropy = -jnp.sum(a * jnp.log(a + 1e-08), axis=-1)
    max_entropy = jnp.log(jnp.asarray(KS, dtype=jnp.float32))
    surprise = entropy / jnp.clip(max_entropy, 1e-08, None)
    importance = magnitude * (1.0 + surprise)
    learned = jnp.sum(hs * w_ref[...], axis=-1) + b_ref[0]
    importance = importance + jax.nn.sigmoid(learned)
    out_ref[...] = importance


def _importance(hs, attn3, w, b):
    return pl.pallas_call(
        _imp_kernel,
        grid=(K1_GRID,),
        in_specs=[
            pl.BlockSpec((K1_BLK, HIDDEN), lambda i: (i, 0)),
            pl.BlockSpec((1, K1_BLK, KS), lambda i: (i, 0, 0)),
            pl.BlockSpec((1, HIDDEN), lambda i: (0, 0)),
            pl.BlockSpec(memory_space=pltpu.SMEM),
        ],
        out_specs=pl.BlockSpec((K1_BLK,), lambda i: (i,)),
        out_shape=jax.ShapeDtypeStruct((T_TOK,), jnp.float32),
    )(hs, attn3, w, b)


def _mask_kernel(imp_ref, maskf_ref, maski_ref):
    imp = imp_ref[...].reshape(64, 128)
    bits = lax.bitcast_convert_type(imp, jnp.int32)
    ids = 128 * lax.broadcasted_iota(jnp.int32, (64, 128), 0) + lax.broadcasted_iota(
        jnp.int32, (64, 128), 1
    )

    def count_ge(v):
        return jnp.sum((bits >= v).astype(jnp.int32))

    def body(_, carry):
        lo, hi = carry
        mid = lo + (hi - lo + 1) // 2
        c = count_ge(mid)
        big = c >= WRITE_TOP_K
        return (jnp.where(big, mid, lo), jnp.where(big, hi, mid - 1))

    lo, _ = lax.fori_loop(0, 31, body, (jnp.int32(0), jnp.int32(2**31 - 2)))
    n_gt = jnp.sum((bits > lo).astype(jnp.int32))
    need = WRITE_TOP_K - n_gt
    eq = bits == lo

    def count_eq_le(j):
        return jnp.sum((eq & (ids <= j)).astype(jnp.int32))

    def body2(_, carry):
        jlo, jhi = carry
        mid = jlo + (jhi - jlo) // 2
        c = count_eq_le(mid)
        ok = c >= need
        return (jnp.where(ok, jlo, mid + 1), jnp.where(ok, mid, jhi))

    jlo, _ = lax.fori_loop(0, 14, body2, (jnp.int32(0), jnp.int32(T_TOK - 1)))
    mask = (bits > lo) | (eq & (ids <= jlo))
    maskf_ref[...] = mask.astype(jnp.float32).reshape(T_TOK)
    maski_ref[...] = mask.astype(jnp.int32).reshape(T_TOK)


def _topk_mask(imp):
    return pl.pallas_call(
        _mask_kernel,
        out_shape=(
            jax.ShapeDtypeStruct((T_TOK,), jnp.float32),
            jax.ShapeDtypeStruct((T_TOK,), jnp.int32),
        ),
    )(imp)


K2B_BLK = 1024
K2B_GRID = T_TOK // K2B_BLK


def _compact_kernel(mask_ref, sel_ref, off_ref):
    step = pl.program_id(0)

    @pl.when(step == 0)
    def _():
        off_ref[0] = 0

    def body(t, off):
        m = mask_ref[t]

        @pl.when(m != 0)
        def _():
            sel_ref[off] = step * K2B_BLK + t

        return off + m

    off_ref[0] = lax.fori_loop(0, K2B_BLK, body, off_ref[0])


def _compact(maski):
    return pl.pallas_call(
        _compact_kernel,
        grid=(K2B_GRID,),
        in_specs=[pl.BlockSpec((K2B_BLK,), lambda i: (i,), memory_space=pltpu.SMEM)],
        out_specs=pl.BlockSpec((WRITE_TOP_K,), lambda i: (0,), memory_space=pltpu.SMEM),
        out_shape=jax.ShapeDtypeStruct((WRITE_TOP_K,), jnp.int32),
        scratch_shapes=[pltpu.SMEM((1,), jnp.int32)],
    )(maski)


def _gather_kernel(sel_ref, hs_ref, si_ref, hout_ref, sout_ref):
    hout_ref[...] = hs_ref[...].astype(jnp.bfloat16)
    sout_ref[...] = si_ref[...]


def _gather(sel, hs3, si3):
    grid_spec = pltpu.PrefetchScalarGridSpec(
        num_scalar_prefetch=1,
        grid=(WRITE_TOP_K,),
        in_specs=[
            pl.BlockSpec((1, 1, HIDDEN), lambda i, sel_ref: (sel_ref[i], 0, 0)),
            pl.BlockSpec((1, 1, KS), lambda i, sel_ref: (sel_ref[i], 0, 0)),
        ],
        out_specs=[
            pl.BlockSpec((1, 1, HIDDEN), lambda i, sel_ref: (i, 0, 0)),
            pl.BlockSpec((1, 1, KS), lambda i, sel_ref: (i, 0, 0)),
        ],
    )
    return pl.pallas_call(
        _gather_kernel,
        grid_spec=grid_spec,
        out_shape=(
            jax.ShapeDtypeStruct((WRITE_TOP_K, 1, HIDDEN), jnp.bfloat16),
            jax.ShapeDtypeStruct((WRITE_TOP_K, 1, KS), jnp.int32),
        ),
    )(sel, hs3, si3)


SB = 512
TB = 128
SGRID = NUM_SLOTS // SB
TGRID = WRITE_TOP_K // TB


def _scatter_kernel(hsel_ref, ssel_ref, mem_ref, out_ref, acc_ref, cnt_ref):
    s = pl.program_id(0)
    t = pl.program_id(1)

    @pl.when(t == 0)
    def _():
        acc_ref[...] = jnp.zeros_like(acc_ref)
        cnt_ref[...] = jnp.zeros_like(cnt_ref)

    idx = ssel_ref[...]
    base = s * SB
    srow = base + lax.broadcasted_iota(jnp.int32, (SB, TB), 0)
    st = jnp.zeros((SB, TB), jnp.float32)
    for k in range(KS):
        st = st + (idx[k : k + 1, :] == srow).astype(jnp.float32)
    st_bf = st.astype(jnp.bfloat16)
    acc_ref[...] += jax.lax.dot_general(
        st_bf,
        hsel_ref[...],
        (((1,), (0,)), ((), ())),
        preferred_element_type=jnp.float32,
    )
    cnt_ref[...] += st

    @pl.when(t == TGRID - 1)
    def _():
        cnt = jnp.sum(cnt_ref[...], axis=1, keepdims=True)
        cnt_bf = cnt.astype(jnp.bfloat16)
        active = cnt_bf > 0
        safe = jnp.where(active, cnt_bf, jnp.asarray(1.0, jnp.bfloat16))
        agg = acc_ref[...].astype(jnp.bfloat16) / safe
        cur = mem_ref[...]
        new = (
            EMA_ALPHA * agg.astype(jnp.float32) + (1.0 - EMA_ALPHA) * cur.astype(jnp.float32)
        ).astype(jnp.bfloat16)
        out_ref[...] = jnp.where(active, new, cur)


def _scatter_ema(hsel, ssel_t, mem2d):
    return pl.pallas_call(
        _scatter_kernel,
        grid=(SGRID, TGRID),
        in_specs=[
            pl.BlockSpec((TB, HIDDEN), lambda s, t: (t, 0)),
            pl.BlockSpec((KS, TB), lambda s, t: (0, t)),
            pl.BlockSpec((SB, HIDDEN), lambda s, t: (s, 0)),
        ],
        out_specs=pl.BlockSpec((SB, HIDDEN), lambda s, t: (s, 0)),
        out_shape=jax.ShapeDtypeStruct((NUM_SLOTS, HIDDEN), jnp.bfloat16),
        scratch_shapes=[
            pltpu.VMEM((SB, HIDDEN), jnp.float32),
            pltpu.VMEM((SB, TB), jnp.float32),
        ],
    )(hsel, ssel_t, mem2d)


def kernel(hidden_states, slot_indices, attention_weights, memory, W_imp, b_imp, batch_idx):
    attn3 = attention_weights.reshape(K1_GRID, K1_BLK, KS)
    imp = _importance(hidden_states, attn3, W_imp, b_imp)
    maskf, maski = _topk_mask(imp)
    del maskf
    sel = _compact(maski)
    si3 = slot_indices.reshape(T_TOK, 1, KS)
    hs3 = hidden_states.reshape(T_TOK, 1, HIDDEN)
    hsel3, ssel = _gather(sel, hs3, si3)
    ssel_t = ssel.reshape(WRITE_TOP_K, KS).T
    new_mem = _scatter_ema(hsel3.reshape(WRITE_TOP_K, HIDDEN), ssel_t, memory[0])
    return lax.dynamic_update_index_in_dim(memory, new_mem, batch_idx, 0)


# speedup vs baseline: 2.2343x; 2.2343x over previous
import functools

import jax
import jax.numpy as jnp
from jax import lax
from jax.experimental import pallas as pl
from jax.experimental.pallas import tpu as pltpu

NUM_SLOTS = 4096
HIDDEN = 4096
T_TOK = 8192
KS = 8
EMA_ALPHA = 0.1
WRITE_TOP_K = 2048

K1_BLK = 256
K1_GRID = T_TOK // K1_BLK


def _imp_kernel(hs_ref, attn_ref, w_ref, b_ref, out_ref):
    hs = hs_ref[...]
    ssq = jnp.sum(hs * hs, axis=-1)
    magnitude = jnp.sqrt(ssq)
    a = attn_ref[0]
    entropy = -jnp.sum(a * jnp.log(a + 1e-08), axis=-1)
    max_entropy = jnp.log(jnp.asarray(KS, dtype=jnp.float32))
    surprise = entropy / jnp.clip(max_entropy, 1e-08, None)
    importance = magnitude * (1.0 + surprise)
    learned = jnp.sum(hs * w_ref[...], axis=-1) + b_ref[0]
    importance = importance + jax.nn.sigmoid(learned)
    out_ref[...] = importance


def _importance(hs, attn3, w, b):
    return pl.pallas_call(
        _imp_kernel,
        grid=(K1_GRID,),
        in_specs=[
            pl.BlockSpec((K1_BLK, HIDDEN), lambda i: (i, 0)),
            pl.BlockSpec((1, K1_BLK, KS), lambda i: (i, 0, 0)),
            pl.BlockSpec((1, HIDDEN), lambda i: (0, 0)),
            pl.BlockSpec(memory_space=pltpu.SMEM),
        ],
        out_specs=pl.BlockSpec((K1_BLK,), lambda i: (i,)),
        out_shape=jax.ShapeDtypeStruct((T_TOK,), jnp.float32),
    )(hs, attn3, w, b)


def _mask_kernel(imp_ref, maskf_ref, maski_ref):
    imp = imp_ref[...].reshape(64, 128)
    bits = lax.bitcast_convert_type(imp, jnp.int32)
    ids = 128 * lax.broadcasted_iota(jnp.int32, (64, 128), 0) + lax.broadcasted_iota(
        jnp.int32, (64, 128), 1
    )

    def count_ge(v):
        return jnp.sum((bits >= v).astype(jnp.int32))

    def body(_, carry):
        lo, hi = carry
        mid = lo + (hi - lo + 1) // 2
        c = count_ge(mid)
        big = c >= WRITE_TOP_K
        return (jnp.where(big, mid, lo), jnp.where(big, hi, mid - 1))

    lo, _ = lax.fori_loop(0, 31, body, (jnp.int32(0), jnp.int32(2**31 - 2)))
    n_gt = jnp.sum((bits > lo).astype(jnp.int32))
    need = WRITE_TOP_K - n_gt
    eq = bits == lo

    def count_eq_le(j):
        return jnp.sum((eq & (ids <= j)).astype(jnp.int32))

    def body2(_, carry):
        jlo, jhi = carry
        mid = jlo + (jhi - jlo) // 2
        c = count_eq_le(mid)
        ok = c >= need
        return (jnp.where(ok, jlo, mid + 1), jnp.where(ok, mid, jhi))

    jlo, _ = lax.fori_loop(0, 14, body2, (jnp.int32(0), jnp.int32(T_TOK - 1)))
    mask = (bits > lo) | (eq & (ids <= jlo))
    maskf_ref[...] = mask.astype(jnp.float32).reshape(T_TOK)
    maski_ref[...] = mask.astype(jnp.int32).reshape(T_TOK)


def _topk_mask(imp):
    return pl.pallas_call(
        _mask_kernel,
        out_shape=(
            jax.ShapeDtypeStruct((T_TOK,), jnp.float32),
            jax.ShapeDtypeStruct((T_TOK,), jnp.int32),
        ),
    )(imp)


K2B_BLK = 1024
K2B_GRID = T_TOK // K2B_BLK


def _compact_kernel(mask_ref, sel_ref, off_ref):
    step = pl.program_id(0)

    @pl.when(step == 0)
    def _():
        off_ref[0] = 0

    def body(t, off):
        m = mask_ref[t]

        @pl.when(m != 0)
        def _():
            sel_ref[off] = step * K2B_BLK + t

        return off + m

    off_ref[0] = lax.fori_loop(0, K2B_BLK, body, off_ref[0])


def _compact(maski):
    return pl.pallas_call(
        _compact_kernel,
        grid=(K2B_GRID,),
        in_specs=[pl.BlockSpec((K2B_BLK,), lambda i: (i,), memory_space=pltpu.SMEM)],
        out_specs=pl.BlockSpec((WRITE_TOP_K,), lambda i: (0,), memory_space=pltpu.SMEM),
        out_shape=jax.ShapeDtypeStruct((WRITE_TOP_K,), jnp.int32),
        scratch_shapes=[pltpu.SMEM((1,), jnp.int32)],
    )(maski)


GW = 8


def _gather_kernel(sel_ref, *refs):
    hrefs = refs[:GW]
    srefs = refs[GW : 2 * GW]
    hout_ref = refs[2 * GW]
    sout_ref = refs[2 * GW + 1]
    for j in range(GW):
        hout_ref[0, j, :] = hrefs[j][0, 0, :].astype(jnp.bfloat16)
        sout_ref[0, j, :] = srefs[j][0, 0, :]


def _make_imap(j):
    return lambda i, sel_ref: (sel_ref[GW * i + j], 0, 0)


def _gather(sel, hs3, si3):
    grid_spec = pltpu.PrefetchScalarGridSpec(
        num_scalar_prefetch=1,
        grid=(WRITE_TOP_K // GW,),
        in_specs=[pl.BlockSpec((1, 1, HIDDEN), _make_imap(j)) for j in range(GW)]
        + [pl.BlockSpec((1, 1, KS), _make_imap(j)) for j in range(GW)],
        out_specs=[
            pl.BlockSpec((1, GW, HIDDEN), lambda i, sel_ref: (i, 0, 0)),
            pl.BlockSpec((1, GW, KS), lambda i, sel_ref: (i, 0, 0)),
        ],
    )
    return pl.pallas_call(
        _gather_kernel,
        grid_spec=grid_spec,
        out_shape=(
            jax.ShapeDtypeStruct((WRITE_TOP_K // GW, GW, HIDDEN), jnp.bfloat16),
            jax.ShapeDtypeStruct((WRITE_TOP_K // GW, GW, KS), jnp.int32),
        ),
    )(sel, *([hs3] * GW), *([si3] * GW))


SB = 512
TB = 128
SGRID = NUM_SLOTS // SB
TGRID = WRITE_TOP_K // TB


def _scatter_kernel(hsel_ref, ssel_ref, mem_ref, out_ref, acc_ref, cnt_ref):
    s = pl.program_id(0)
    t = pl.program_id(1)

    @pl.when(t == 0)
    def _():
        acc_ref[...] = jnp.zeros_like(acc_ref)
        cnt_ref[...] = jnp.zeros_like(cnt_ref)

    idx = ssel_ref[...]
    base = s * SB
    srow = base + lax.broadcasted_iota(jnp.int32, (SB, TB), 0)
    st = jnp.zeros((SB, TB), jnp.float32)
    for k in range(KS):
        st = st + (idx[k : k + 1, :] == srow).astype(jnp.float32)
    st_bf = st.astype(jnp.bfloat16)
    acc_ref[...] += jax.lax.dot_general(
        st_bf,
        hsel_ref[...],
        (((1,), (0,)), ((), ())),
        preferred_element_type=jnp.float32,
    )
    cnt_ref[...] += st

    @pl.when(t == TGRID - 1)
    def _():
        cnt = jnp.sum(cnt_ref[...], axis=1, keepdims=True)
        cnt_bf = cnt.astype(jnp.bfloat16)
        active = cnt_bf > 0
        safe = jnp.where(active, cnt_bf, jnp.asarray(1.0, jnp.bfloat16))
        agg = acc_ref[...].astype(jnp.bfloat16) / safe
        cur = mem_ref[...]
        new = (
            EMA_ALPHA * agg.astype(jnp.float32) + (1.0 - EMA_ALPHA) * cur.astype(jnp.float32)
        ).astype(jnp.bfloat16)
        out_ref[...] = jnp.where(active, new, cur)


def _scatter_ema(hsel, ssel_t, mem2d):
    return pl.pallas_call(
        _scatter_kernel,
        grid=(SGRID, TGRID),
        in_specs=[
            pl.BlockSpec((TB, HIDDEN), lambda s, t: (t, 0)),
            pl.BlockSpec((KS, TB), lambda s, t: (0, t)),
            pl.BlockSpec((SB, HIDDEN), lambda s, t: (s, 0)),
        ],
        out_specs=pl.BlockSpec((SB, HIDDEN), lambda s, t: (s, 0)),
        out_shape=jax.ShapeDtypeStruct((NUM_SLOTS, HIDDEN), jnp.bfloat16),
        scratch_shapes=[
            pltpu.VMEM((SB, HIDDEN), jnp.float32),
            pltpu.VMEM((SB, TB), jnp.float32),
        ],
    )(hsel, ssel_t, mem2d)


def kernel(hidden_states, slot_indices, attention_weights, memory, W_imp, b_imp, batch_idx):
    attn3 = attention_weights.reshape(K1_GRID, K1_BLK, KS)
    imp = _importance(hidden_states, attn3, W_imp, b_imp)
    maskf, maski = _topk_mask(imp)
    del maskf
    sel = _compact(maski)
    si3 = slot_indices.reshape(T_TOK, 1, KS)
    hs3 = hidden_states.reshape(T_TOK, 1, HIDDEN)
    hsel3, ssel = _gather(sel, hs3, si3)
    ssel_t = ssel.reshape(WRITE_TOP_K, KS).T
    new_mem = _scatter_ema(hsel3.reshape(WRITE_TOP_K, HIDDEN), ssel_t, memory[0])
    return lax.dynamic_update_index_in_dim(memory, new_mem, batch_idx, 0)
